# baseline (speedup 1.0000x reference)
import functools

import jax
import jax.numpy as jnp
from jax import lax
from jax.experimental import pallas as pl
from jax.experimental.pallas import tpu as pltpu
from jax.experimental.pallas import tpu_sc as plsc


def _proj_body(x_ref, wr_ref, ws_ref, xr_ref, xs_ref):
    xb = x_ref[...]
    xr_ref[...] = jnp.dot(xb, wr_ref[...], preferred_element_type=jnp.float32)
    xs_ref[...] = jnp.dot(xb, ws_ref[...], preferred_element_type=jnp.float32)


def _eb_body(ea_ref, we_ref, b_ref, eb_ref):
    y = (
        jnp.dot(ea_ref[...], we_ref[...], preferred_element_type=jnp.float32)
        + b_ref[...]
    )
    ya = y.reshape(y.shape[0] // 2, 2, y.shape[1])
    ue = lax.bitcast_convert_type(
        ya[:, 0, :].astype(jnp.bfloat16), jnp.uint16
    ).astype(jnp.uint32)
    uo = lax.bitcast_convert_type(
        ya[:, 1, :].astype(jnp.bfloat16), jnp.uint16
    ).astype(jnp.uint32)
    eb_ref[...] = lax.bitcast_convert_type(ue | (uo << 16), jnp.int32)


def kernel(x, edge_attr, W, b, edge_index):
    N, F = x.shape
    E, DE = edge_attr.shape
    DO = W.shape[1]

    We = W[:DE]
    Wr = W[DE : DE + F]
    Ws = W[DE + F :]
    idx = edge_index.astype(jnp.int32)
    idx_r = idx[0]
    idx_s = idx[1]
    b2 = b.reshape(1, DO)

    BN = 1000
    xr, xs = pl.pallas_call(
        _proj_body,
        grid=(N // BN,),
        in_specs=[
            pl.BlockSpec((BN, F), lambda i: (i, 0)),
            pl.BlockSpec((F, DO), lambda i: (0, 0)),
            pl.BlockSpec((F, DO), lambda i: (0, 0)),
        ],
        out_specs=[
            pl.BlockSpec((BN, DO), lambda i: (i, 0)),
            pl.BlockSpec((BN, DO), lambda i: (i, 0)),
        ],
        out_shape=[
            jax.ShapeDtypeStruct((N, DO), jnp.float32),
            jax.ShapeDtypeStruct((N, DO), jnp.float32),
        ],
    )(x, Wr, Ws)

    BE = 16000
    eb = pl.pallas_call(
        _eb_body,
        grid=(E // BE,),
        in_specs=[
            pl.BlockSpec((BE, DE), lambda i: (i, 0)),
            pl.BlockSpec((DE, DO), lambda i: (0, 0)),
            pl.BlockSpec((1, DO), lambda i: (0, 0)),
        ],
        out_specs=pl.BlockSpec((BE // 2, DO), lambda i: (i, 0)),
        out_shape=jax.ShapeDtypeStruct((E // 2, DO), jnp.int32),
    )(edge_attr, We, b2)

    info = plsc.get_sparse_core_info()
    NC, NS = info.num_cores, info.num_subcores
    NW = NC * NS
    per_w = E // NW
    C = 128
    n_full = per_w // C
    rem = per_w - n_full * C
    n_pairs = n_full // 2
    odd_tail = n_full - 2 * n_pairs
    CR = max(rem, 8)

    mesh = plsc.VectorSubcoreMesh(core_axis_name="c", subcore_axis_name="s")

    @functools.partial(
        pl.kernel,
        mesh=mesh,
        out_type=jax.ShapeDtypeStruct((E, DO), jnp.float32),
        scratch_types=[
            pltpu.VMEM((C,), jnp.int32),
            pltpu.VMEM((C,), jnp.int32),
            pltpu.VMEM((C // 2, DO), jnp.int32),
            pltpu.VMEM((C, DO), jnp.float32),
            pltpu.VMEM((C,), jnp.int32),
            pltpu.VMEM((C,), jnp.int32),
            pltpu.VMEM((C // 2, DO), jnp.int32),
            pltpu.VMEM((C, DO), jnp.float32),
            pltpu.VMEM((CR,), jnp.int32),
            pltpu.VMEM((CR,), jnp.int32),
            pltpu.VMEM((CR // 2, DO), jnp.int32),
            pltpu.VMEM((CR, DO), jnp.float32),
            pltpu.SemaphoreType.DMA,
            pltpu.SemaphoreType.DMA,
            pltpu.SemaphoreType.DMA,
            pltpu.SemaphoreType.DMA,
            pltpu.SemaphoreType.DMA,
        ],
    )
    def _sc_combine(
        idx_r_hbm, idx_s_hbm, xr_hbm, xs_hbm, eb_hbm, out_hbm,
        idxr0, idxs0, ebb0, acc0,
        idxr1, idxs1, ebb1, acc1,
        idxr_t, idxs_t, ebb_t, acc_t,
        lsem0, lsem1, osem0, osem1, gsem,
    ):
        wid = lax.axis_index("s") * NC + lax.axis_index("c")
        w_base = wid * per_w
        idxr = (idxr0, idxr1)
        idxs = (idxs0, idxs1)
        ebb = (ebb0, ebb1)
        acc = (acc0, acc1)
        lsem = (lsem0, lsem1)
        osem = (osem0, osem1)

        MASK = jnp.int32(-65536)

        def loads(ci, slot):
            base = w_base + ci * C
            base2 = wid * (per_w // 2) + ci * (C // 2)
            return (
                pltpu.make_async_copy(
                    idx_r_hbm.at[pl.ds(base, C)], idxr[slot], lsem[slot]),
                pltpu.make_async_copy(
                    idx_s_hbm.at[pl.ds(base, C)], idxs[slot], lsem[slot]),
                pltpu.make_async_copy(
                    eb_hbm.at[pl.ds(base2, C // 2)], ebb[slot], lsem[slot]),
            )

        def store(ci, slot):
            base = w_base + ci * C
            return pltpu.make_async_copy(
                acc[slot], out_hbm.at[pl.ds(base, C)], osem[slot])

        def seed(n_packed, ebb_b, acc_b):
            def row_body(rp, carry):
                r2 = 2 * rp
                for g in range(8):
                    sl = pl.ds(16 * g, 16)
                    w = ebb_b[rp, sl]
                    acc_b[r2, sl] = lax.bitcast_convert_type(
                        w << 16, jnp.float32)
                    acc_b[r2 + 1, sl] = lax.bitcast_convert_type(
                        w & MASK, jnp.float32)
                return carry

            lax.fori_loop(0, n_packed, row_body, 0)

        def chunk_step(ci, slot):
            g1 = pltpu.async_copy(xr_hbm.at[idxr[slot]], acc[slot], gsem,
                                  add=True)
            g2 = pltpu.async_copy(xs_hbm.at[idxs[slot]], acc[slot], gsem,
                                  add=True)

            @pl.when(ci + 1 < n_full)
            def _():
                @pl.when(ci >= 1)
                def _():
                    store(ci - 1, 1 - slot).wait()

                for d in loads(ci + 1, 1 - slot):
                    d.wait()
                seed(C // 2, ebb[1 - slot], acc[1 - slot])

            g1.wait()
            g2.wait()
            @pl.when(ci + 2 < n_full)
            def _():
                for d in loads(ci + 2, slot):
                    d.start()

            store(ci, slot).start()

        for d in loads(0, 0):
            d.start()
        for d in loads(1, 1):
            d.start()
        for d in loads(0, 0):
            d.wait()
        seed(C // 2, ebb0, acc0)

        def pair_body(p, carry):
            for bslot in range(2):
                chunk_step(2 * p + bslot, bslot)
            return carry

        lax.fori_loop(0, n_pairs, pair_body, 0)

        if odd_tail:
            chunk_step(2 * n_pairs, 0)

        last = n_full - 1
        store(last - 1, (last - 1) % 2).wait()
        store(last, last % 2).wait()

        if rem:
            base = w_base + n_full * C
            pltpu.sync_copy(idx_r_hbm.at[pl.ds(base, rem)], idxr_t)
            pltpu.sync_copy(idx_s_hbm.at[pl.ds(base, rem)], idxs_t)
            pltpu.sync_copy(
                eb_hbm.at[pl.ds(wid * (per_w // 2) + n_full * (C // 2),
                                rem // 2)], ebb_t)
            seed(rem // 2, ebb_t, acc_t)
            g1 = pltpu.async_copy(xr_hbm.at[idxr_t], acc_t, gsem, add=True)
            g2 = pltpu.async_copy(xs_hbm.at[idxs_t], acc_t, gsem, add=True)
            g1.wait()
            g2.wait()
            pltpu.sync_copy(acc_t, out_hbm.at[pl.ds(base, rem)])

    return _sc_combine(idx_r, idx_s, xr, xs, eb)


# speedup vs baseline: 1.2566x; 1.2566x over previous
import functools

import jax
import jax.numpy as jnp
from jax import lax
from jax.experimental import pallas as pl
from jax.experimental.pallas import tpu as pltpu
from jax.experimental.pallas import tpu_sc as plsc


def _proj_body(x_ref, wr_ref, ws_ref, xr_ref, xs_ref):
    xb = x_ref[...]
    xr_ref[...] = jnp.dot(xb, wr_ref[...], preferred_element_type=jnp.float32)
    xs_ref[...] = jnp.dot(xb, ws_ref[...], preferred_element_type=jnp.float32)


def _eb_body(ea_ref, we_ref, b_ref, eb_ref):
    eb_ref[...] = (
        jnp.dot(ea_ref[...], we_ref[...], preferred_element_type=jnp.float32)
        + b_ref[...]
    )


def kernel(x, edge_attr, W, b, edge_index):
    N, F = x.shape
    E, DE = edge_attr.shape
    DO = W.shape[1]

    We = W[:DE]
    Wr = W[DE : DE + F]
    Ws = W[DE + F :]
    idx = edge_index.astype(jnp.int32)
    idx_r = idx[0]
    idx_s = idx[1]
    b2 = b.reshape(1, DO)

    BN = 1000
    xr, xs = pl.pallas_call(
        _proj_body,
        grid=(N // BN,),
        in_specs=[
            pl.BlockSpec((BN, F), lambda i: (i, 0)),
            pl.BlockSpec((F, DO), lambda i: (0, 0)),
            pl.BlockSpec((F, DO), lambda i: (0, 0)),
        ],
        out_specs=[
            pl.BlockSpec((BN, DO), lambda i: (i, 0)),
            pl.BlockSpec((BN, DO), lambda i: (i, 0)),
        ],
        out_shape=[
            jax.ShapeDtypeStruct((N, DO), jnp.float32),
            jax.ShapeDtypeStruct((N, DO), jnp.float32),
        ],
    )(x, Wr, Ws)

    BE = 16000
    eb = pl.pallas_call(
        _eb_body,
        grid=(E // BE,),
        in_specs=[
            pl.BlockSpec((BE, DE), lambda i: (i, 0)),
            pl.BlockSpec((DE, DO), lambda i: (0, 0)),
            pl.BlockSpec((1, DO), lambda i: (0, 0)),
        ],
        out_specs=pl.BlockSpec((BE, DO), lambda i: (i, 0)),
        out_shape=jax.ShapeDtypeStruct((E, DO), jnp.float32),
    )(edge_attr, We, b2)

    info = plsc.get_sparse_core_info()
    NC, NS = info.num_cores, info.num_subcores
    NW = NC * NS
    per_w = E // NW
    C = 128
    n_full = per_w // C
    rem = per_w - n_full * C
    n_pairs = n_full // 2
    odd_tail = n_full - 2 * n_pairs

    mesh = plsc.VectorSubcoreMesh(core_axis_name="c", subcore_axis_name="s")

    @functools.partial(
        pl.kernel,
        mesh=mesh,
        out_type=jax.ShapeDtypeStruct((E, DO), jnp.float32),
        scratch_types=[
            pltpu.VMEM((C,), jnp.int32),
            pltpu.VMEM((C,), jnp.int32),
            pltpu.VMEM((C, DO), jnp.float32),
            pltpu.VMEM((C,), jnp.int32),
            pltpu.VMEM((C,), jnp.int32),
            pltpu.VMEM((C, DO), jnp.float32),
            pltpu.VMEM((max(rem, 8),), jnp.int32),
            pltpu.VMEM((max(rem, 8),), jnp.int32),
            pltpu.VMEM((max(rem, 8), DO), jnp.float32),
            pltpu.SemaphoreType.DMA,
            pltpu.SemaphoreType.DMA,
            pltpu.SemaphoreType.DMA,
            pltpu.SemaphoreType.DMA,
            pltpu.SemaphoreType.DMA,
        ],
    )
    def _sc_combine(
        idx_r_hbm, idx_s_hbm, xr_hbm, xs_hbm, eb_hbm, out_hbm,
        idxr0, idxs0, acc0, idxr1, idxs1, acc1, idxr_t, idxs_t, acc_t,
        lsem0, lsem1, osem0, osem1, gsem,
    ):
        wid = lax.axis_index("s") * NC + lax.axis_index("c")
        w_base = wid * per_w
        idxr = (idxr0, idxr1)
        idxs = (idxs0, idxs1)
        acc = (acc0, acc1)
        lsem = (lsem0, lsem1)
        osem = (osem0, osem1)

        def loads(ci, slot):
            base = w_base + ci * C
            return (
                pltpu.make_async_copy(
                    idx_r_hbm.at[pl.ds(base, C)], idxr[slot], lsem[slot]),
                pltpu.make_async_copy(
                    idx_s_hbm.at[pl.ds(base, C)], idxs[slot], lsem[slot]),
                pltpu.make_async_copy(
                    eb_hbm.at[pl.ds(base, C)], acc[slot], lsem[slot]),
            )

        def store(ci, slot):
            base = w_base + ci * C
            return pltpu.make_async_copy(
                acc[slot], out_hbm.at[pl.ds(base, C)], osem[slot])

        def issue(descs):
            for d in descs:
                d.start()

        def chunk_step(ci, slot):
            @pl.when(ci >= 1)
            def _():
                store(ci - 1, 1 - slot).wait()

            @pl.when(ci + 1 < n_full)
            def _():
                issue(loads(ci + 1, 1 - slot))

            d1 = pltpu.async_copy(
                xr_hbm.at[idxr[slot]], acc[slot], gsem, add=True)
            d2 = pltpu.async_copy(
                xs_hbm.at[idxs[slot]], acc[slot], gsem, add=True)
            d1.wait()
            d2.wait()
            store(ci, slot).start()

        issue(loads(0, 0))

        def pair_body(p, carry):
            for b in range(2):
                ci = 2 * p + b
                for d in loads(ci, b):
                    d.wait()
                chunk_step(ci, b)
            return carry

        lax.fori_loop(0, n_pairs, pair_body, 0)

        if odd_tail:
            ci = 2 * n_pairs
            for d in loads(ci, 0):
                d.wait()
            chunk_step(ci, 0)

        last = n_full - 1
        store(last, last % 2).wait()

        if rem:
            base = w_base + n_full * C
            pltpu.sync_copy(idx_r_hbm.at[pl.ds(base, rem)], idxr_t)
            pltpu.sync_copy(idx_s_hbm.at[pl.ds(base, rem)], idxs_t)
            pltpu.sync_copy(eb_hbm.at[pl.ds(base, rem)], acc_t)
            d1 = pltpu.async_copy(xr_hbm.at[idxr_t], acc_t, gsem, add=True)
            d2 = pltpu.async_copy(xs_hbm.at[idxs_t], acc_t, gsem, add=True)
            d1.wait()
            d2.wait()
            pltpu.sync_copy(acc_t, out_hbm.at[pl.ds(base, rem)])

    return _sc_combine(idx_r, idx_s, xr, xs, eb)


# speedup vs baseline: 1.6461x; 1.3099x over previous
import functools

import jax
import jax.numpy as jnp
from jax import lax
from jax.experimental import pallas as pl
from jax.experimental.pallas import tpu as pltpu
from jax.experimental.pallas import tpu_sc as plsc


def _proj_body(x_ref, wr_ref, ws_ref, xr_ref, xs_ref):
    xb = x_ref[...]
    xr_ref[...] = jnp.dot(xb, wr_ref[...], preferred_element_type=jnp.float32)
    xs_ref[...] = jnp.dot(xb, ws_ref[...], preferred_element_type=jnp.float32)


def _eb_body(ea_t_ref, we_ref, b_ref, eb_ref):
    eb_ref[...] = (
        lax.dot_general(
            ea_t_ref[...], we_ref[...], (((0,), (0,)), ((), ())),
            preferred_element_type=jnp.float32,
        )
        + b_ref[...]
    )


def kernel(x, edge_attr, W, b, edge_index):
    N, F = x.shape
    E, DE = edge_attr.shape
    DO = W.shape[1]

    We = W[:DE]
    Wr = W[DE : DE + F]
    Ws = W[DE + F :]
    idx = edge_index.astype(jnp.int32)
    idx_r = idx[0]
    idx_s = idx[1]
    b2 = b.reshape(1, DO)
    ea_t = edge_attr.T

    BN = 1000
    xr, xs = pl.pallas_call(
        _proj_body,
        grid=(N // BN,),
        in_specs=[
            pl.BlockSpec((BN, F), lambda i: (i, 0)),
            pl.BlockSpec((F, DO), lambda i: (0, 0)),
            pl.BlockSpec((F, DO), lambda i: (0, 0)),
        ],
        out_specs=[
            pl.BlockSpec((BN, DO), lambda i: (i, 0)),
            pl.BlockSpec((BN, DO), lambda i: (i, 0)),
        ],
        out_shape=[
            jax.ShapeDtypeStruct((N, DO), jnp.float32),
            jax.ShapeDtypeStruct((N, DO), jnp.float32),
        ],
    )(x, Wr, Ws)

    BE = 16000
    eb = pl.pallas_call(
        _eb_body,
        grid=(E // BE,),
        in_specs=[
            pl.BlockSpec((DE, BE), lambda i: (0, i)),
            pl.BlockSpec((DE, DO), lambda i: (0, 0)),
            pl.BlockSpec((1, DO), lambda i: (0, 0)),
        ],
        out_specs=pl.BlockSpec((BE, DO), lambda i: (i, 0)),
        out_shape=jax.ShapeDtypeStruct((E, DO), jnp.float32),
    )(ea_t, We, b2)

    info = plsc.get_sparse_core_info()
    NC, NS = info.num_cores, info.num_subcores
    NW = NC * NS
    per_w = E // NW
    C = 128
    n_full = per_w // C
    rem = per_w - n_full * C
    n_pairs = n_full // 2
    odd_tail = n_full - 2 * n_pairs

    mesh = plsc.VectorSubcoreMesh(core_axis_name="c", subcore_axis_name="s")

    @functools.partial(
        pl.kernel,
        mesh=mesh,
        out_type=jax.ShapeDtypeStruct((E, DO), jnp.float32),
        scratch_types=[
            pltpu.VMEM((C,), jnp.int32),
            pltpu.VMEM((C,), jnp.int32),
            pltpu.VMEM((C, DO), jnp.float32),
            pltpu.VMEM((C,), jnp.int32),
            pltpu.VMEM((C,), jnp.int32),
            pltpu.VMEM((C, DO), jnp.float32),
            pltpu.VMEM((max(rem, 8),), jnp.int32),
            pltpu.VMEM((max(rem, 8),), jnp.int32),
            pltpu.VMEM((max(rem, 8), DO), jnp.float32),
            pltpu.SemaphoreType.DMA,
            pltpu.SemaphoreType.DMA,
            pltpu.SemaphoreType.DMA,
            pltpu.SemaphoreType.DMA,
            pltpu.SemaphoreType.DMA,
        ],
    )
    def _sc_combine(
        idx_r_hbm, idx_s_hbm, xr_hbm, xs_hbm, eb_hbm, out_hbm,
        idxr0, idxs0, acc0, idxr1, idxs1, acc1, idxr_t, idxs_t, acc_t,
        lsem0, lsem1, osem0, osem1, gsem,
    ):
        wid = lax.axis_index("s") * NC + lax.axis_index("c")
        w_base = wid * per_w
        idxr = (idxr0, idxr1)
        idxs = (idxs0, idxs1)
        acc = (acc0, acc1)
        lsem = (lsem0, lsem1)
        osem = (osem0, osem1)

        def loads(ci, slot):
            base = w_base + ci * C
            return (
                pltpu.make_async_copy(
                    idx_r_hbm.at[pl.ds(base, C)], idxr[slot], lsem[slot]),
                pltpu.make_async_copy(
                    idx_s_hbm.at[pl.ds(base, C)], idxs[slot], lsem[slot]),
                pltpu.make_async_copy(
                    eb_hbm.at[pl.ds(base, C)], acc[slot], lsem[slot]),
            )

        def store(ci, slot):
            base = w_base + ci * C
            return pltpu.make_async_copy(
                acc[slot], out_hbm.at[pl.ds(base, C)], osem[slot])

        def issue(descs):
            for d in descs:
                d.start()

        def chunk_step(ci, slot):
            @pl.when(ci >= 1)
            def _():
                store(ci - 1, 1 - slot).wait()

            @pl.when(ci + 1 < n_full)
            def _():
                issue(loads(ci + 1, 1 - slot))

            d1 = pltpu.async_copy(
                xr_hbm.at[idxr[slot]], acc[slot], gsem, add=True)
            d2 = pltpu.async_copy(
                xs_hbm.at[idxs[slot]], acc[slot], gsem, add=True)
            d1.wait()
            d2.wait()
            store(ci, slot).start()

        issue(loads(0, 0))

        def pair_body(p, carry):
            for b in range(2):
                ci = 2 * p + b
                for d in loads(ci, b):
                    d.wait()
                chunk_step(ci, b)
            return carry

        lax.fori_loop(0, n_pairs, pair_body, 0)

        if odd_tail:
            ci = 2 * n_pairs
            for d in loads(ci, 0):
                d.wait()
            chunk_step(ci, 0)

        last = n_full - 1
        store(last, last % 2).wait()

        if rem:
            base = w_base + n_full * C
            pltpu.sync_copy(idx_r_hbm.at[pl.ds(base, rem)], idxr_t)
            pltpu.sync_copy(idx_s_hbm.at[pl.ds(base, rem)], idxs_t)
            pltpu.sync_copy(eb_hbm.at[pl.ds(base, rem)], acc_t)
            d1 = pltpu.async_copy(xr_hbm.at[idxr_t], acc_t, gsem, add=True)
            d2 = pltpu.async_copy(xs_hbm.at[idxs_t], acc_t, gsem, add=True)
            d1.wait()
            d2.wait()
            pltpu.sync_copy(acc_t, out_hbm.at[pl.ds(base, rem)])

    return _sc_combine(idx_r, idx_s, xr, xs, eb)


# speedup vs baseline: 1.6988x; 1.0320x over previous
import functools

import jax
import jax.numpy as jnp
from jax import lax
from jax.experimental import pallas as pl
from jax.experimental.pallas import tpu as pltpu
from jax.experimental.pallas import tpu_sc as plsc


def _proj_body(x_ref, wr_ref, ws_ref, xr_ref, xs_ref):
    xb = x_ref[...]
    xr_ref[...] = jnp.dot(xb, wr_ref[...], preferred_element_type=jnp.float32)
    xs_ref[...] = jnp.dot(xb, ws_ref[...], preferred_element_type=jnp.float32)


def _eb_body(ea_t_ref, we_ref, b_ref, eb_ref):
    eb_ref[...] = (
        lax.dot_general(
            ea_t_ref[...], we_ref[...], (((0,), (0,)), ((), ())),
            preferred_element_type=jnp.float32,
        )
        + b_ref[...]
    )


def kernel(x, edge_attr, W, b, edge_index):
    N, F = x.shape
    E, DE = edge_attr.shape
    DO = W.shape[1]

    We = W[:DE]
    Wr = W[DE : DE + F]
    Ws = W[DE + F :]
    idx = edge_index.astype(jnp.int32)
    idx_r = idx[0]
    idx_s = idx[1]
    b2 = b.reshape(1, DO)
    ea_t = edge_attr.T

    BN = 1000
    xr, xs = pl.pallas_call(
        _proj_body,
        grid=(N // BN,),
        in_specs=[
            pl.BlockSpec((BN, F), lambda i: (i, 0)),
            pl.BlockSpec((F, DO), lambda i: (0, 0)),
            pl.BlockSpec((F, DO), lambda i: (0, 0)),
        ],
        out_specs=[
            pl.BlockSpec((BN, DO), lambda i: (i, 0)),
            pl.BlockSpec((BN, DO), lambda i: (i, 0)),
        ],
        out_shape=[
            jax.ShapeDtypeStruct((N, DO), jnp.float32),
            jax.ShapeDtypeStruct((N, DO), jnp.float32),
        ],
    )(x, Wr, Ws)

    BE = 16000
    eb = pl.pallas_call(
        _eb_body,
        grid=(E // BE,),
        in_specs=[
            pl.BlockSpec((DE, BE), lambda i: (0, i)),
            pl.BlockSpec((DE, DO), lambda i: (0, 0)),
            pl.BlockSpec((1, DO), lambda i: (0, 0)),
        ],
        out_specs=pl.BlockSpec((BE, DO), lambda i: (i, 0)),
        out_shape=jax.ShapeDtypeStruct((E, DO), jnp.float32),
    )(ea_t, We, b2)

    info = plsc.get_sparse_core_info()
    NC, NS = info.num_cores, info.num_subcores
    NW = NC * NS
    per_w = E // NW
    C = 256
    n_full = per_w // C
    rem = per_w - n_full * C
    n_pairs = n_full // 2
    odd_tail = n_full - 2 * n_pairs

    mesh = plsc.VectorSubcoreMesh(core_axis_name="c", subcore_axis_name="s")

    @functools.partial(
        pl.kernel,
        mesh=mesh,
        out_type=jax.ShapeDtypeStruct((E, DO), jnp.float32),
        scratch_types=[
            pltpu.VMEM((C,), jnp.int32),
            pltpu.VMEM((C,), jnp.int32),
            pltpu.VMEM((C, DO), jnp.float32),
            pltpu.VMEM((C,), jnp.int32),
            pltpu.VMEM((C,), jnp.int32),
            pltpu.VMEM((C, DO), jnp.float32),
            pltpu.VMEM((max(rem, 8),), jnp.int32),
            pltpu.VMEM((max(rem, 8),), jnp.int32),
            pltpu.VMEM((max(rem, 8), DO), jnp.float32),
            pltpu.SemaphoreType.DMA,
            pltpu.SemaphoreType.DMA,
            pltpu.SemaphoreType.DMA,
            pltpu.SemaphoreType.DMA,
            pltpu.SemaphoreType.DMA,
        ],
    )
    def _sc_combine(
        idx_r_hbm, idx_s_hbm, xr_hbm, xs_hbm, eb_hbm, out_hbm,
        idxr0, idxs0, acc0, idxr1, idxs1, acc1, idxr_t, idxs_t, acc_t,
        lsem0, lsem1, osem0, osem1, gsem,
    ):
        wid = lax.axis_index("s") * NC + lax.axis_index("c")
        w_base = wid * per_w
        idxr = (idxr0, idxr1)
        idxs = (idxs0, idxs1)
        acc = (acc0, acc1)
        lsem = (lsem0, lsem1)
        osem = (osem0, osem1)

        def loads(ci, slot):
            base = w_base + ci * C
            return (
                pltpu.make_async_copy(
                    idx_r_hbm.at[pl.ds(base, C)], idxr[slot], lsem[slot]),
                pltpu.make_async_copy(
                    idx_s_hbm.at[pl.ds(base, C)], idxs[slot], lsem[slot]),
                pltpu.make_async_copy(
                    eb_hbm.at[pl.ds(base, C)], acc[slot], lsem[slot]),
            )

        def store(ci, slot):
            base = w_base + ci * C
            return pltpu.make_async_copy(
                acc[slot], out_hbm.at[pl.ds(base, C)], osem[slot])

        def issue(descs):
            for d in descs:
                d.start()

        def chunk_step(ci, slot):
            @pl.when(ci >= 1)
            def _():
                store(ci - 1, 1 - slot).wait()

            @pl.when(ci + 1 < n_full)
            def _():
                issue(loads(ci + 1, 1 - slot))

            descs = []
            for lo in (0, C // 2):
                sl = pl.ds(lo, C // 2)
                descs.append(pltpu.async_copy(
                    xr_hbm.at[idxr[slot].at[sl]], acc[slot].at[sl], gsem,
                    add=True))
                descs.append(pltpu.async_copy(
                    xs_hbm.at[idxs[slot].at[sl]], acc[slot].at[sl], gsem,
                    add=True))
            for d in descs:
                d.wait()
            store(ci, slot).start()

        issue(loads(0, 0))

        def pair_body(p, carry):
            for b in range(2):
                ci = 2 * p + b
                for d in loads(ci, b):
                    d.wait()
                chunk_step(ci, b)
            return carry

        lax.fori_loop(0, n_pairs, pair_body, 0)

        if odd_tail:
            ci = 2 * n_pairs
            for d in loads(ci, 0):
                d.wait()
            chunk_step(ci, 0)

        last = n_full - 1
        store(last, last % 2).wait()

        if rem:
            base = w_base + n_full * C
            pltpu.sync_copy(idx_r_hbm.at[pl.ds(base, rem)], idxr_t)
            pltpu.sync_copy(idx_s_hbm.at[pl.ds(base, rem)], idxs_t)
            pltpu.sync_copy(eb_hbm.at[pl.ds(base, rem)], acc_t)
            d1 = pltpu.async_copy(xr_hbm.at[idxr_t], acc_t, gsem, add=True)
            d2 = pltpu.async_copy(xs_hbm.at[idxs_t], acc_t, gsem, add=True)
            d1.wait()
            d2.wait()
            pltpu.sync_copy(acc_t, out_hbm.at[pl.ds(base, rem)])

    return _sc_combine(idx_r, idx_s, xr, xs, eb)


# speedup vs baseline: 1.7330x; 1.0201x over previous
import functools

import jax
import jax.numpy as jnp
from jax import lax
from jax.experimental import pallas as pl
from jax.experimental.pallas import tpu as pltpu
from jax.experimental.pallas import tpu_sc as plsc


def _proj_body(x_ref, wr_ref, ws_ref, xr_ref, xs_ref):
    xb = x_ref[...]
    xr_ref[...] = jnp.dot(xb, wr_ref[...], preferred_element_type=jnp.float32)
    xs_ref[...] = jnp.dot(xb, ws_ref[...], preferred_element_type=jnp.float32)


def _eb_body(ea_t_ref, we_ref, b_ref, eb_ref):
    eb_ref[...] = (
        lax.dot_general(
            ea_t_ref[...], we_ref[...], (((0,), (0,)), ((), ())),
            preferred_element_type=jnp.float32,
        )
        + b_ref[...]
    )


def kernel(x, edge_attr, W, b, edge_index):
    N, F = x.shape
    E, DE = edge_attr.shape
    DO = W.shape[1]

    We = W[:DE]
    Wr = W[DE : DE + F]
    Ws = W[DE + F :]
    idx = edge_index.astype(jnp.int32)
    idx_r = idx[0]
    idx_s = idx[1]
    b2 = b.reshape(1, DO)
    ea_t = edge_attr.T

    BN = 1000
    xr, xs = pl.pallas_call(
        _proj_body,
        grid=(N // BN,),
        in_specs=[
            pl.BlockSpec((BN, F), lambda i: (i, 0)),
            pl.BlockSpec((F, DO), lambda i: (0, 0)),
            pl.BlockSpec((F, DO), lambda i: (0, 0)),
        ],
        out_specs=[
            pl.BlockSpec((BN, DO), lambda i: (i, 0)),
            pl.BlockSpec((BN, DO), lambda i: (i, 0)),
        ],
        out_shape=[
            jax.ShapeDtypeStruct((N, DO), jnp.float32),
            jax.ShapeDtypeStruct((N, DO), jnp.float32),
        ],
    )(x, Wr, Ws)

    BE = 16000
    eb = pl.pallas_call(
        _eb_body,
        grid=(E // BE,),
        in_specs=[
            pl.BlockSpec((DE, BE), lambda i: (0, i)),
            pl.BlockSpec((DE, DO), lambda i: (0, 0)),
            pl.BlockSpec((1, DO), lambda i: (0, 0)),
        ],
        out_specs=pl.BlockSpec((BE, DO), lambda i: (i, 0)),
        out_shape=jax.ShapeDtypeStruct((E, DO), jnp.float32),
    )(ea_t, We, b2)

    info = plsc.get_sparse_core_info()
    NC, NS = info.num_cores, info.num_subcores
    NW = NC * NS
    per_w = E // NW
    C = 384
    n_full = per_w // C
    rem = per_w - n_full * C
    n_pairs = n_full // 2
    odd_tail = n_full - 2 * n_pairs

    mesh = plsc.VectorSubcoreMesh(core_axis_name="c", subcore_axis_name="s")

    @functools.partial(
        pl.kernel,
        mesh=mesh,
        out_type=jax.ShapeDtypeStruct((E, DO), jnp.float32),
        scratch_types=[
            pltpu.VMEM((C,), jnp.int32),
            pltpu.VMEM((C,), jnp.int32),
            pltpu.VMEM((C, DO), jnp.float32),
            pltpu.VMEM((C,), jnp.int32),
            pltpu.VMEM((C,), jnp.int32),
            pltpu.VMEM((C, DO), jnp.float32),
            pltpu.VMEM((max(rem, 8),), jnp.int32),
            pltpu.VMEM((max(rem, 8),), jnp.int32),
            pltpu.VMEM((max(rem, 8), DO), jnp.float32),
            pltpu.SemaphoreType.DMA,
            pltpu.SemaphoreType.DMA,
            pltpu.SemaphoreType.DMA,
            pltpu.SemaphoreType.DMA,
            pltpu.SemaphoreType.DMA,
        ],
    )
    def _sc_combine(
        idx_r_hbm, idx_s_hbm, xr_hbm, xs_hbm, eb_hbm, out_hbm,
        idxr0, idxs0, acc0, idxr1, idxs1, acc1, idxr_t, idxs_t, acc_t,
        lsem0, lsem1, osem0, osem1, gsem,
    ):
        wid = lax.axis_index("s") * NC + lax.axis_index("c")
        w_base = wid * per_w
        idxr = (idxr0, idxr1)
        idxs = (idxs0, idxs1)
        acc = (acc0, acc1)
        lsem = (lsem0, lsem1)
        osem = (osem0, osem1)

        def loads(ci, slot):
            base = w_base + ci * C
            return (
                pltpu.make_async_copy(
                    idx_r_hbm.at[pl.ds(base, C)], idxr[slot], lsem[slot]),
                pltpu.make_async_copy(
                    idx_s_hbm.at[pl.ds(base, C)], idxs[slot], lsem[slot]),
                pltpu.make_async_copy(
                    eb_hbm.at[pl.ds(base, C)], acc[slot], lsem[slot]),
            )

        def store(ci, slot):
            base = w_base + ci * C
            return pltpu.make_async_copy(
                acc[slot], out_hbm.at[pl.ds(base, C)], osem[slot])

        def issue(descs):
            for d in descs:
                d.start()

        def chunk_step(ci, slot):
            @pl.when(ci >= 1)
            def _():
                store(ci - 1, 1 - slot).wait()

            @pl.when(ci + 1 < n_full)
            def _():
                issue(loads(ci + 1, 1 - slot))

            descs = []
            for lo in (0, C // 3, 2 * (C // 3)):
                sl = pl.ds(lo, C // 3)
                descs.append(pltpu.async_copy(
                    xr_hbm.at[idxr[slot].at[sl]], acc[slot].at[sl], gsem,
                    add=True))
                descs.append(pltpu.async_copy(
                    xs_hbm.at[idxs[slot].at[sl]], acc[slot].at[sl], gsem,
                    add=True))
            for d in descs:
                d.wait()
            store(ci, slot).start()

        issue(loads(0, 0))

        def pair_body(p, carry):
            for b in range(2):
                ci = 2 * p + b
                for d in loads(ci, b):
                    d.wait()
                chunk_step(ci, b)
            return carry

        lax.fori_loop(0, n_pairs, pair_body, 0)

        if odd_tail:
            ci = 2 * n_pairs
            for d in loads(ci, 0):
                d.wait()
            chunk_step(ci, 0)

        last = n_full - 1
        store(last, last % 2).wait()

        if rem:
            base = w_base + n_full * C
            pltpu.sync_copy(idx_r_hbm.at[pl.ds(base, rem)], idxr_t)
            pltpu.sync_copy(idx_s_hbm.at[pl.ds(base, rem)], idxs_t)
            pltpu.sync_copy(eb_hbm.at[pl.ds(base, rem)], acc_t)
            d1 = pltpu.async_copy(xr_hbm.at[idxr_t], acc_t, gsem, add=True)
            d2 = pltpu.async_copy(xs_hbm.at[idxs_t], acc_t, gsem, add=True)
            d1.wait()
            d2.wait()
            pltpu.sync_copy(acc_t, out_hbm.at[pl.ds(base, rem)])

    return _sc_combine(idx_r, idx_s, xr, xs, eb)
